# baseline (speedup 1.0000x reference)
import functools
import math

import jax
import jax.numpy as jnp
from jax import lax
from jax.experimental import pallas as pl
from jax.experimental.pallas import tpu as pltpu
from jax.experimental.pallas import tpu_sc as plsc

_N = 10000
_E = 320000
_D = 128
_H = 128
_C = 40
_L = 8
_ALPHA = 0.1
_LAMDA = 0.5

_NC = 2
_NS = 16
_NT = _NC * _NS

_NP = 10240
_RPT = _NP // _NS
_K = 128
_G = 16
_GROUPS = 5
_CHUNKS = _G * _GROUPS
_EP = _NT * _CHUNKS * _K

_BR = 512
_GRID = _NP // _BR


def _mesh():
    return plsc.VectorSubcoreMesh(core_axis_name="c", subcore_axis_name="s")


def _sc_degree(srcp):
    @functools.partial(
        pl.kernel,
        out_type=jax.ShapeDtypeStruct((_NC, _NP), jnp.float32),
        mesh=_mesh(),
        scratch_types=[
            pltpu.VMEM_SHARED((_NP,), jnp.float32),
            pltpu.VMEM((_K,), jnp.int32),
            pltpu.VMEM((_K,), jnp.float32),
            pltpu.VMEM((_RPT,), jnp.float32),
        ],
    )
    def deg_kernel(src_hbm, out_hbm, cnt_sh, idx_v, ones_v, bounce_v):
        c = lax.axis_index("c")
        s = lax.axis_index("s")
        w = s * _NC + c
        for j in range(_K // 16):
            ones_v[pl.ds(j * 16, 16)] = jnp.full((16,), 1.0, jnp.float32)
        for j in range(_RPT // 16):
            bounce_v[pl.ds(j * 16, 16)] = jnp.zeros((16,), jnp.float32)
        pltpu.sync_copy(bounce_v, cnt_sh.at[pl.ds(s * _RPT, _RPT)])
        plsc.subcore_barrier()
        base = w * (_CHUNKS * _K)

        def body(j, carry):
            off = base + j * _K
            pltpu.sync_copy(src_hbm.at[pl.ds(off, _K)], idx_v)
            pltpu.sync_copy(ones_v, cnt_sh.at[idx_v], add=True)
            return carry

        lax.fori_loop(0, _CHUNKS, body, 0)
        plsc.subcore_barrier()
        pltpu.sync_copy(cnt_sh.at[pl.ds(s * _RPT, _RPT)], bounce_v)
        pltpu.sync_copy(bounce_v, out_hbm.at[c, pl.ds(s * _RPT, _RPT)])

    return deg_kernel(srcp)


def _sc_spmm(g, srcp, dstp):
    @functools.partial(
        pl.kernel,
        out_type=jax.ShapeDtypeStruct((_NC, _NP, _H), jnp.float32),
        mesh=_mesh(),
        scratch_types=[
            pltpu.VMEM_SHARED((_NP, _H), jnp.float32),
            pltpu.VMEM((_K,), jnp.int32),
            pltpu.VMEM((_K,), jnp.int32),
            pltpu.VMEM((_K, _H), jnp.float32),
            pltpu.SemaphoreType.DMA,
        ],
    )
    def spmm_kernel(g_hbm, src_hbm, dst_hbm, out_hbm, acc_sh,
                    idxs_v, idxd_v, rows_v, sem):
        c = lax.axis_index("c")
        s = lax.axis_index("s")
        w = s * _NC + c

        def zbody(i, carry):
            for j in range(_H // 16):
                rows_v[i, pl.ds(j * 16, 16)] = jnp.zeros((16,), jnp.float32)
            return carry

        lax.fori_loop(0, _K, zbody, 0)
        r0 = s * _RPT
        for rep in range(_RPT // _K):
            pltpu.sync_copy(rows_v, acc_sh.at[pl.ds(r0 + rep * _K, _K)])
        plsc.subcore_barrier()

        base = w * (_CHUNKS * _K)

        def body(j, carry):
            off = base + j * _K
            pltpu.sync_copy(src_hbm.at[pl.ds(off, _K)], idxs_v)
            pltpu.sync_copy(dst_hbm.at[pl.ds(off, _K)], idxd_v)
            pltpu.async_copy(g_hbm.at[idxs_v], rows_v, sem).wait()
            pltpu.sync_copy(rows_v, acc_sh.at[idxd_v], add=True)
            return carry

        lax.fori_loop(0, _CHUNKS, body, 0)
        plsc.subcore_barrier()
        for rep in range(_RPT // _K):
            rr = r0 + rep * _K
            pltpu.sync_copy(acc_sh.at[pl.ds(rr, _K)], rows_v)
            pltpu.sync_copy(rows_v, out_hbm.at[c, pl.ds(rr, _K)])

    return spmm_kernel(g, srcp, dstp)


def _tc_input(xp, counts_t, w_in, b_in):
    def body(x_ref, cnt_ref, w_ref, b_ref, h0_ref, g_ref, dis_ref):
        h = jnp.dot(x_ref[...], w_ref[...], preferred_element_type=jnp.float32)
        h = jnp.maximum(h + b_ref[...], 0.0)
        deg = 1.0 + cnt_ref[:, 0:1] + cnt_ref[:, 1:2]
        dis = lax.rsqrt(deg)
        h0_ref[...] = h
        g_ref[...] = h * dis
        dis_ref[...] = dis

    return pl.pallas_call(
        body,
        grid=(_GRID,),
        in_specs=[
            pl.BlockSpec((_BR, _D), lambda i: (i, 0)),
            pl.BlockSpec((_BR, 2), lambda i: (i, 0)),
            pl.BlockSpec((_D, _H), lambda i: (0, 0)),
            pl.BlockSpec((1, _H), lambda i: (0, 0)),
        ],
        out_specs=[
            pl.BlockSpec((_BR, _H), lambda i: (i, 0)),
            pl.BlockSpec((_BR, _H), lambda i: (i, 0)),
            pl.BlockSpec((_BR, 1), lambda i: (i, 0)),
        ],
        out_shape=[
            jax.ShapeDtypeStruct((_NP, _H), jnp.float32),
            jax.ShapeDtypeStruct((_NP, _H), jnp.float32),
            jax.ShapeDtypeStruct((_NP, 1), jnp.float32),
        ],
    )(xp, counts_t, w_in, b_in)


def _tc_layer(acc, h0, g, dis, w, b, beta_arr):
    def body(beta_ref, acc_ref, h0_ref, g_ref, dis_ref, w_ref, b_ref,
             h_ref, gout_ref):
        asum = acc_ref[0] + acc_ref[1] + g_ref[...]
        dis_b = dis_ref[...]
        hi = asum * dis_b
        support = (1.0 - _ALPHA) * hi + _ALPHA * h0_ref[...]
        t = jnp.dot(support, w_ref[...], preferred_element_type=jnp.float32)
        beta = beta_ref[0]
        out = beta * t + (1.0 - beta) * support + b_ref[...]
        h = jnp.maximum(out, 0.0)
        h_ref[...] = h
        gout_ref[...] = h * dis_b

    return pl.pallas_call(
        body,
        grid=(_GRID,),
        in_specs=[
            pl.BlockSpec(memory_space=pltpu.SMEM),
            pl.BlockSpec((_NC, _BR, _H), lambda i: (0, i, 0)),
            pl.BlockSpec((_BR, _H), lambda i: (i, 0)),
            pl.BlockSpec((_BR, _H), lambda i: (i, 0)),
            pl.BlockSpec((_BR, 1), lambda i: (i, 0)),
            pl.BlockSpec((_H, _H), lambda i: (0, 0)),
            pl.BlockSpec((1, _H), lambda i: (0, 0)),
        ],
        out_specs=[
            pl.BlockSpec((_BR, _H), lambda i: (i, 0)),
            pl.BlockSpec((_BR, _H), lambda i: (i, 0)),
        ],
        out_shape=[
            jax.ShapeDtypeStruct((_NP, _H), jnp.float32),
            jax.ShapeDtypeStruct((_NP, _H), jnp.float32),
        ],
    )(beta_arr, acc, h0, g, dis, w, b)


def _tc_out(h, w_out, b_out):
    grid = -(-_N // _BR)

    def body(h_ref, w_ref, b_ref, o_ref):
        o_ref[...] = (
            jnp.dot(h_ref[...], w_ref[...], preferred_element_type=jnp.float32)
            + b_ref[...]
        )

    return pl.pallas_call(
        body,
        grid=(grid,),
        in_specs=[
            pl.BlockSpec((_BR, _H), lambda i: (i, 0)),
            pl.BlockSpec((_H, _C), lambda i: (0, 0)),
            pl.BlockSpec((1, _C), lambda i: (0, 0)),
        ],
        out_specs=pl.BlockSpec((_BR, _C), lambda i: (i, 0)),
        out_shape=jax.ShapeDtypeStruct((_N, _C), jnp.float32),
    )(h, w_out, b_out)


def kernel(x, edge_index, W_in, b_in, Wl, bl, W_out, b_out):
    src = edge_index[0]
    dst = edge_index[1]
    pad = jnp.full((_EP - _E,), _N, jnp.int32)
    srcp = jnp.concatenate([src, pad])
    dstp = jnp.concatenate([dst, pad])
    xp = jnp.pad(x, ((0, _NP - _N), (0, 0)))

    counts = _sc_degree(srcp)
    h0, g, dis = _tc_input(xp, counts.T, W_in, b_in.reshape(1, _H))
    h = h0
    for i in range(_L):
        beta = math.log(_LAMDA / (i + 1) + 1.0)
        acc = _sc_spmm(g, srcp, dstp)
        h, g = _tc_layer(acc, h0, g, dis, Wl[i], bl[i].reshape(1, _H),
                         jnp.array([beta], jnp.float32))
    return _tc_out(h, W_out, b_out.reshape(1, _C))


# speedup vs baseline: 2.3628x; 2.3628x over previous
import functools
import math

import jax
import jax.numpy as jnp
from jax import lax
from jax.experimental import pallas as pl
from jax.experimental.pallas import tpu as pltpu
from jax.experimental.pallas import tpu_sc as plsc

_N = 10000
_E = 320000
_D = 128
_H = 128
_C = 40
_L = 8
_ALPHA = 0.1
_LAMDA = 0.5

_NC = 2
_NS = 16
_NT = _NC * _NS

_NP = 10240
_RPT = _NP // _NS
_K = 128
_G = 16
_GROUPS = 5
_CHUNKS = _G * _GROUPS
_EP = _NT * _CHUNKS * _K

_BR = 512
_GRID = _NP // _BR


def _mesh():
    return plsc.VectorSubcoreMesh(core_axis_name="c", subcore_axis_name="s")


def _sc_degree(srcp):
    @functools.partial(
        pl.kernel,
        out_type=jax.ShapeDtypeStruct((_NC, _NP), jnp.float32),
        mesh=_mesh(),
        scratch_types=[
            pltpu.VMEM_SHARED((_NP,), jnp.float32),
            pltpu.VMEM((_K,), jnp.int32),
            pltpu.VMEM((_K,), jnp.float32),
            pltpu.VMEM((_RPT,), jnp.float32),
        ],
    )
    def deg_kernel(src_hbm, out_hbm, cnt_sh, idx_v, ones_v, bounce_v):
        c = lax.axis_index("c")
        s = lax.axis_index("s")
        w = s * _NC + c
        for j in range(_K // 16):
            ones_v[pl.ds(j * 16, 16)] = jnp.full((16,), 1.0, jnp.float32)
        for j in range(_RPT // 16):
            bounce_v[pl.ds(j * 16, 16)] = jnp.zeros((16,), jnp.float32)
        pltpu.sync_copy(bounce_v, cnt_sh.at[pl.ds(s * _RPT, _RPT)])
        plsc.subcore_barrier()
        base = w * (_CHUNKS * _K)

        def body(j, carry):
            off = base + j * _K
            pltpu.sync_copy(src_hbm.at[pl.ds(off, _K)], idx_v)
            pltpu.sync_copy(ones_v, cnt_sh.at[idx_v], add=True)
            return carry

        lax.fori_loop(0, _CHUNKS, body, 0)
        plsc.subcore_barrier()
        pltpu.sync_copy(cnt_sh.at[pl.ds(s * _RPT, _RPT)], bounce_v)
        pltpu.sync_copy(bounce_v, out_hbm.at[c, pl.ds(s * _RPT, _RPT)])

    return deg_kernel(srcp)


def _sc_spmm(g, srcp, dstp):
    @functools.partial(
        pl.kernel,
        out_type=jax.ShapeDtypeStruct((_NC, _NP, _H), jnp.float32),
        mesh=_mesh(),
        scratch_types=[
            pltpu.VMEM_SHARED((_NP, _H), jnp.float32),
            pltpu.VMEM((_K,), jnp.int32),
            pltpu.VMEM((_K,), jnp.int32),
            pltpu.VMEM((_K, _H), jnp.float32),
            pltpu.SemaphoreType.DMA,
        ],
    )
    def spmm_kernel(g_hbm, src_hbm, dst_hbm, out_hbm, acc_sh,
                    idxs_v, idxd_v, rows_v, sem):
        c = lax.axis_index("c")
        s = lax.axis_index("s")
        w = s * _NC + c

        def zbody(i, carry):
            for j in range(_H // 16):
                rows_v[i, pl.ds(j * 16, 16)] = jnp.zeros((16,), jnp.float32)
            return carry

        lax.fori_loop(0, _K, zbody, 0)
        r0 = s * _RPT
        for rep in range(_RPT // _K):
            pltpu.sync_copy(rows_v, acc_sh.at[pl.ds(r0 + rep * _K, _K)])
        plsc.subcore_barrier()

        base = w * (_CHUNKS * _K)

        def body(j, carry):
            off = base + j * _K
            pltpu.sync_copy(src_hbm.at[pl.ds(off, _K)], idxs_v)
            pltpu.sync_copy(dst_hbm.at[pl.ds(off, _K)], idxd_v)
            pltpu.async_copy(g_hbm.at[idxs_v], rows_v, sem).wait()
            pltpu.sync_copy(rows_v, acc_sh.at[idxd_v], add=True)
            return carry

        lax.fori_loop(0, _CHUNKS, body, 0)
        plsc.subcore_barrier()
        for rep in range(_RPT // _K):
            rr = r0 + rep * _K
            pltpu.sync_copy(acc_sh.at[pl.ds(rr, _K)], rows_v)
            pltpu.sync_copy(rows_v, out_hbm.at[c, pl.ds(rr, _K)])

    return spmm_kernel(g, srcp, dstp)


def _tc_input(xp, counts_t, w_in, b_in):
    def body(x_ref, cnt_ref, w_ref, b_ref, h0_ref, g_ref, dis_ref):
        h = jnp.dot(x_ref[...], w_ref[...], preferred_element_type=jnp.float32)
        h = jnp.maximum(h + b_ref[...], 0.0)
        deg = 1.0 + cnt_ref[:, 0:1] + cnt_ref[:, 1:2]
        dis = lax.rsqrt(deg)
        h0_ref[...] = h
        g_ref[...] = h * dis
        dis_ref[...] = dis

    return pl.pallas_call(
        body,
        grid=(_GRID,),
        in_specs=[
            pl.BlockSpec((_BR, _D), lambda i: (i, 0)),
            pl.BlockSpec((_BR, 2), lambda i: (i, 0)),
            pl.BlockSpec((_D, _H), lambda i: (0, 0)),
            pl.BlockSpec((1, _H), lambda i: (0, 0)),
        ],
        out_specs=[
            pl.BlockSpec((_BR, _H), lambda i: (i, 0)),
            pl.BlockSpec((_BR, _H), lambda i: (i, 0)),
            pl.BlockSpec((_BR, 1), lambda i: (i, 0)),
        ],
        out_shape=[
            jax.ShapeDtypeStruct((_NP, _H), jnp.float32),
            jax.ShapeDtypeStruct((_NP, _H), jnp.float32),
            jax.ShapeDtypeStruct((_NP, 1), jnp.float32),
        ],
    )(xp, counts_t, w_in, b_in)


def _tc_layer(acc, h0, g, dis, w, b, beta_arr):
    def body(beta_ref, acc_ref, h0_ref, g_ref, dis_ref, w_ref, b_ref,
             h_ref, gout_ref):
        asum = acc_ref[0] + acc_ref[1] + g_ref[...]
        dis_b = dis_ref[...]
        hi = asum * dis_b
        support = (1.0 - _ALPHA) * hi + _ALPHA * h0_ref[...]
        t = jnp.dot(support, w_ref[...], preferred_element_type=jnp.float32)
        beta = beta_ref[0]
        out = beta * t + (1.0 - beta) * support + b_ref[...]
        h = jnp.maximum(out, 0.0)
        h_ref[...] = h
        gout_ref[...] = h * dis_b

    return pl.pallas_call(
        body,
        grid=(_GRID,),
        in_specs=[
            pl.BlockSpec(memory_space=pltpu.SMEM),
            pl.BlockSpec((_NC, _BR, _H), lambda i: (0, i, 0)),
            pl.BlockSpec((_BR, _H), lambda i: (i, 0)),
            pl.BlockSpec((_BR, _H), lambda i: (i, 0)),
            pl.BlockSpec((_BR, 1), lambda i: (i, 0)),
            pl.BlockSpec((_H, _H), lambda i: (0, 0)),
            pl.BlockSpec((1, _H), lambda i: (0, 0)),
        ],
        out_specs=[
            pl.BlockSpec((_BR, _H), lambda i: (i, 0)),
            pl.BlockSpec((_BR, _H), lambda i: (i, 0)),
        ],
        out_shape=[
            jax.ShapeDtypeStruct((_NP, _H), jnp.float32),
            jax.ShapeDtypeStruct((_NP, _H), jnp.float32),
        ],
    )(beta_arr, acc, h0, g, dis, w, b)


def _tc_out(h, w_out, b_out):
    grid = -(-_N // _BR)

    def body(h_ref, w_ref, b_ref, o_ref):
        o_ref[...] = (
            jnp.dot(h_ref[...], w_ref[...], preferred_element_type=jnp.float32)
            + b_ref[...]
        )

    return pl.pallas_call(
        body,
        grid=(grid,),
        in_specs=[
            pl.BlockSpec((_BR, _H), lambda i: (i, 0)),
            pl.BlockSpec((_H, _C), lambda i: (0, 0)),
            pl.BlockSpec((1, _C), lambda i: (0, 0)),
        ],
        out_specs=pl.BlockSpec((_BR, _C), lambda i: (i, 0)),
        out_shape=jax.ShapeDtypeStruct((_N, _C), jnp.float32),
    )(h, w_out, b_out)


def kernel(x, edge_index, W_in, b_in, Wl, bl, W_out, b_out):
    src = edge_index[0]
    dst = edge_index[1]
    pad = _N + (jnp.arange(_EP - _E, dtype=jnp.int32) % (_NP - _N))
    srcp = jnp.concatenate([src, pad])
    dstp = jnp.concatenate([dst, pad])
    xp = jnp.pad(x, ((0, _NP - _N), (0, 0)))

    counts = _sc_degree(srcp)
    h0, g, dis = _tc_input(xp, counts.T, W_in, b_in.reshape(1, _H))
    h = h0
    for i in range(_L):
        beta = math.log(_LAMDA / (i + 1) + 1.0)
        acc = _sc_spmm(g, srcp, dstp)
        h, g = _tc_layer(acc, h0, g, dis, Wl[i], bl[i].reshape(1, _H),
                         jnp.array([beta], jnp.float32))
    return _tc_out(h, W_out, b_out.reshape(1, _C))


# speedup vs baseline: 3.1906x; 1.3503x over previous
import functools
import math

import jax
import jax.numpy as jnp
from jax import lax
from jax.experimental import pallas as pl
from jax.experimental.pallas import tpu as pltpu
from jax.experimental.pallas import tpu_sc as plsc

_N = 10000
_E = 320000
_D = 128
_H = 128
_C = 40
_L = 8
_ALPHA = 0.1
_LAMDA = 0.5

_NC = 2
_NS = 16
_NT = _NC * _NS

_NP = 10240
_RPT = _NP // _NS
_K = 128
_G = 16
_GROUPS = 5
_CHUNKS = _G * _GROUPS
_EP = _NT * _CHUNKS * _K

_BR = 512
_GRID = _NP // _BR


def _mesh():
    return plsc.VectorSubcoreMesh(core_axis_name="c", subcore_axis_name="s")


def _sc_degree(srcp):
    @functools.partial(
        pl.kernel,
        out_type=jax.ShapeDtypeStruct((_NC, _NP), jnp.float32),
        mesh=_mesh(),
        scratch_types=[
            pltpu.VMEM_SHARED((_NP,), jnp.float32),
            pltpu.VMEM((_K,), jnp.int32),
            pltpu.VMEM((_K,), jnp.float32),
            pltpu.VMEM((_RPT,), jnp.float32),
        ],
    )
    def deg_kernel(src_hbm, out_hbm, cnt_sh, idx_v, ones_v, bounce_v):
        c = lax.axis_index("c")
        s = lax.axis_index("s")
        w = s * _NC + c
        for j in range(_K // 16):
            ones_v[pl.ds(j * 16, 16)] = jnp.full((16,), 1.0, jnp.float32)
        for j in range(_RPT // 16):
            bounce_v[pl.ds(j * 16, 16)] = jnp.zeros((16,), jnp.float32)
        pltpu.sync_copy(bounce_v, cnt_sh.at[pl.ds(s * _RPT, _RPT)])
        plsc.subcore_barrier()
        base = w * (_CHUNKS * _K)

        def body(j, carry):
            off = base + j * _K
            pltpu.sync_copy(src_hbm.at[pl.ds(off, _K)], idx_v)
            pltpu.sync_copy(ones_v, cnt_sh.at[idx_v], add=True)
            return carry

        lax.fori_loop(0, _CHUNKS, body, 0)
        plsc.subcore_barrier()
        pltpu.sync_copy(cnt_sh.at[pl.ds(s * _RPT, _RPT)], bounce_v)
        pltpu.sync_copy(bounce_v, out_hbm.at[c, pl.ds(s * _RPT, _RPT)])

    return deg_kernel(srcp)


def _sc_spmm(g, srcp, dstp):
    @functools.partial(
        pl.kernel,
        out_type=jax.ShapeDtypeStruct((_NC, _NP, _H), jnp.float32),
        mesh=_mesh(),
        scratch_types=[
            pltpu.VMEM_SHARED((_NP, _H), jnp.float32),
            pltpu.VMEM((_K,), jnp.int32),
            pltpu.VMEM((_K,), jnp.int32),
            pltpu.VMEM((_K,), jnp.int32),
            pltpu.VMEM((_K,), jnp.int32),
            pltpu.VMEM((_K, _H), jnp.float32),
            pltpu.VMEM((_K, _H), jnp.float32),
            pltpu.SemaphoreType.DMA,
            pltpu.SemaphoreType.DMA,
        ],
    )
    def spmm_kernel(g_hbm, src_hbm, dst_hbm, out_hbm, acc_sh,
                    idxs_v, idxd_v, idxs_b, idxd_b, rows_v, rows_b,
                    sem, sem_b):
        c = lax.axis_index("c")
        s = lax.axis_index("s")
        w = s * _NC + c

        def zbody(i, carry):
            for j in range(_H // 16):
                rows_v[i, pl.ds(j * 16, 16)] = jnp.zeros((16,), jnp.float32)
            return carry

        lax.fori_loop(0, _K, zbody, 0)
        r0 = s * _RPT
        for rep in range(_RPT // _K):
            pltpu.sync_copy(rows_v, acc_sh.at[pl.ds(r0 + rep * _K, _K)])
        plsc.subcore_barrier()

        base = w * (_CHUNKS * _K)

        def body(t, carry):
            offa = base + (2 * t) * _K
            offb = offa + _K
            pltpu.sync_copy(src_hbm.at[pl.ds(offa, _K)], idxs_v)
            pltpu.sync_copy(dst_hbm.at[pl.ds(offa, _K)], idxd_v)
            da = pltpu.async_copy(g_hbm.at[idxs_v], rows_v, sem)
            pltpu.sync_copy(src_hbm.at[pl.ds(offb, _K)], idxs_b)
            pltpu.sync_copy(dst_hbm.at[pl.ds(offb, _K)], idxd_b)
            db = pltpu.async_copy(g_hbm.at[idxs_b], rows_b, sem_b)
            da.wait()
            pltpu.sync_copy(rows_v, acc_sh.at[idxd_v], add=True)
            db.wait()
            pltpu.sync_copy(rows_b, acc_sh.at[idxd_b], add=True)
            return carry

        lax.fori_loop(0, _CHUNKS // 2, body, 0)
        plsc.subcore_barrier()
        for rep in range(_RPT // _K):
            rr = r0 + rep * _K
            pltpu.sync_copy(acc_sh.at[pl.ds(rr, _K)], rows_v)
            pltpu.sync_copy(rows_v, out_hbm.at[c, pl.ds(rr, _K)])

    return spmm_kernel(g, srcp, dstp)


def _tc_input(xp, counts_t, w_in, b_in):
    def body(x_ref, cnt_ref, w_ref, b_ref, h0_ref, g_ref, dis_ref):
        h = jnp.dot(x_ref[...], w_ref[...], preferred_element_type=jnp.float32)
        h = jnp.maximum(h + b_ref[...], 0.0)
        deg = 1.0 + cnt_ref[:, 0:1] + cnt_ref[:, 1:2]
        dis = lax.rsqrt(deg)
        h0_ref[...] = h
        g_ref[...] = h * dis
        dis_ref[...] = dis

    return pl.pallas_call(
        body,
        grid=(_GRID,),
        in_specs=[
            pl.BlockSpec((_BR, _D), lambda i: (i, 0)),
            pl.BlockSpec((_BR, 2), lambda i: (i, 0)),
            pl.BlockSpec((_D, _H), lambda i: (0, 0)),
            pl.BlockSpec((1, _H), lambda i: (0, 0)),
        ],
        out_specs=[
            pl.BlockSpec((_BR, _H), lambda i: (i, 0)),
            pl.BlockSpec((_BR, _H), lambda i: (i, 0)),
            pl.BlockSpec((_BR, 1), lambda i: (i, 0)),
        ],
        out_shape=[
            jax.ShapeDtypeStruct((_NP, _H), jnp.float32),
            jax.ShapeDtypeStruct((_NP, _H), jnp.float32),
            jax.ShapeDtypeStruct((_NP, 1), jnp.float32),
        ],
    )(xp, counts_t, w_in, b_in)


def _tc_layer(acc, h0, g, dis, w, b, beta_arr):
    def body(beta_ref, acc_ref, h0_ref, g_ref, dis_ref, w_ref, b_ref,
             h_ref, gout_ref):
        asum = acc_ref[0] + acc_ref[1] + g_ref[...]
        dis_b = dis_ref[...]
        hi = asum * dis_b
        support = (1.0 - _ALPHA) * hi + _ALPHA * h0_ref[...]
        t = jnp.dot(support, w_ref[...], preferred_element_type=jnp.float32)
        beta = beta_ref[0]
        out = beta * t + (1.0 - beta) * support + b_ref[...]
        h = jnp.maximum(out, 0.0)
        h_ref[...] = h
        gout_ref[...] = h * dis_b

    return pl.pallas_call(
        body,
        grid=(_GRID,),
        in_specs=[
            pl.BlockSpec(memory_space=pltpu.SMEM),
            pl.BlockSpec((_NC, _BR, _H), lambda i: (0, i, 0)),
            pl.BlockSpec((_BR, _H), lambda i: (i, 0)),
            pl.BlockSpec((_BR, _H), lambda i: (i, 0)),
            pl.BlockSpec((_BR, 1), lambda i: (i, 0)),
            pl.BlockSpec((_H, _H), lambda i: (0, 0)),
            pl.BlockSpec((1, _H), lambda i: (0, 0)),
        ],
        out_specs=[
            pl.BlockSpec((_BR, _H), lambda i: (i, 0)),
            pl.BlockSpec((_BR, _H), lambda i: (i, 0)),
        ],
        out_shape=[
            jax.ShapeDtypeStruct((_NP, _H), jnp.float32),
            jax.ShapeDtypeStruct((_NP, _H), jnp.float32),
        ],
    )(beta_arr, acc, h0, g, dis, w, b)


def _tc_out(h, w_out, b_out):
    grid = -(-_N // _BR)

    def body(h_ref, w_ref, b_ref, o_ref):
        o_ref[...] = (
            jnp.dot(h_ref[...], w_ref[...], preferred_element_type=jnp.float32)
            + b_ref[...]
        )

    return pl.pallas_call(
        body,
        grid=(grid,),
        in_specs=[
            pl.BlockSpec((_BR, _H), lambda i: (i, 0)),
            pl.BlockSpec((_H, _C), lambda i: (0, 0)),
            pl.BlockSpec((1, _C), lambda i: (0, 0)),
        ],
        out_specs=pl.BlockSpec((_BR, _C), lambda i: (i, 0)),
        out_shape=jax.ShapeDtypeStruct((_N, _C), jnp.float32),
    )(h, w_out, b_out)


def kernel(x, edge_index, W_in, b_in, Wl, bl, W_out, b_out):
    src = edge_index[0]
    dst = edge_index[1]
    pad = _N + (jnp.arange(_EP - _E, dtype=jnp.int32) % (_NP - _N))
    srcp = jnp.concatenate([src, pad])
    dstp = jnp.concatenate([dst, pad])
    xp = jnp.pad(x, ((0, _NP - _N), (0, 0)))

    counts = _sc_degree(srcp)
    h0, g, dis = _tc_input(xp, counts.T, W_in, b_in.reshape(1, _H))
    h = h0
    for i in range(_L):
        beta = math.log(_LAMDA / (i + 1) + 1.0)
        acc = _sc_spmm(g, srcp, dstp)
        h, g = _tc_layer(acc, h0, g, dis, Wl[i], bl[i].reshape(1, _H),
                         jnp.array([beta], jnp.float32))
    return _tc_out(h, W_out, b_out.reshape(1, _C))


# speedup vs baseline: 3.5601x; 1.1158x over previous
import functools
import math

import jax
import jax.numpy as jnp
from jax import lax
from jax.experimental import pallas as pl
from jax.experimental.pallas import tpu as pltpu
from jax.experimental.pallas import tpu_sc as plsc

_N = 10000
_E = 320000
_D = 128
_H = 128
_C = 40
_L = 8
_ALPHA = 0.1
_LAMDA = 0.5

_NC = 2
_NS = 16
_NT = _NC * _NS

_NP = 10240
_RPT = _NP // _NS
_K = 128
_G = 16
_GROUPS = 5
_CHUNKS = _G * _GROUPS
_EP = _NT * _CHUNKS * _K

_BR = 512
_GRID = _NP // _BR


def _mesh():
    return plsc.VectorSubcoreMesh(core_axis_name="c", subcore_axis_name="s")


def _sc_degree(srcp):
    @functools.partial(
        pl.kernel,
        out_type=jax.ShapeDtypeStruct((_NC, _NP), jnp.float32),
        mesh=_mesh(),
        scratch_types=[
            pltpu.VMEM_SHARED((_NP,), jnp.float32),
            pltpu.VMEM((_K,), jnp.int32),
            pltpu.VMEM((_K,), jnp.float32),
            pltpu.VMEM((_RPT,), jnp.float32),
        ],
    )
    def deg_kernel(src_hbm, out_hbm, cnt_sh, idx_v, ones_v, bounce_v):
        c = lax.axis_index("c")
        s = lax.axis_index("s")
        w = s * _NC + c
        for j in range(_K // 16):
            ones_v[pl.ds(j * 16, 16)] = jnp.full((16,), 1.0, jnp.float32)
        for j in range(_RPT // 16):
            bounce_v[pl.ds(j * 16, 16)] = jnp.zeros((16,), jnp.float32)
        pltpu.sync_copy(bounce_v, cnt_sh.at[pl.ds(s * _RPT, _RPT)])
        plsc.subcore_barrier()
        base = w * (_CHUNKS * _K)

        def body(j, carry):
            off = base + j * _K
            pltpu.sync_copy(src_hbm.at[pl.ds(off, _K)], idx_v)
            pltpu.sync_copy(ones_v, cnt_sh.at[idx_v], add=True)
            return carry

        lax.fori_loop(0, _CHUNKS, body, 0)
        plsc.subcore_barrier()
        pltpu.sync_copy(cnt_sh.at[pl.ds(s * _RPT, _RPT)], bounce_v)
        pltpu.sync_copy(bounce_v, out_hbm.at[c, pl.ds(s * _RPT, _RPT)])

    return deg_kernel(srcp)


def _sc_spmm(g, srcp, dstp):
    _HC = _CHUNKS // 2

    @functools.partial(
        pl.kernel,
        out_type=jax.ShapeDtypeStruct((_NC, _NP, _H), jnp.float32),
        mesh=_mesh(),
        scratch_types=[
            pltpu.VMEM_SHARED((_NP, _H), jnp.float32),
            pltpu.VMEM((_HC, _K), jnp.int32),
            pltpu.VMEM((_HC, _K), jnp.int32),
            pltpu.VMEM((_K, _H), jnp.float32),
            pltpu.VMEM((_K, _H), jnp.float32),
            pltpu.SemaphoreType.DMA,
            pltpu.SemaphoreType.DMA,
        ],
    )
    def spmm_kernel(g_hbm, src_hbm, dst_hbm, out_hbm, acc_sh,
                    idxs_v, idxd_v, rows_v, rows_b, sem, sem_b):
        c = lax.axis_index("c")
        s = lax.axis_index("s")
        w = s * _NC + c

        def zbody(i, carry):
            for j in range(_H // 16):
                rows_v[i, pl.ds(j * 16, 16)] = jnp.zeros((16,), jnp.float32)
            return carry

        lax.fori_loop(0, _K, zbody, 0)
        r0 = s * _RPT
        for rep in range(_RPT // _K):
            pltpu.sync_copy(rows_v, acc_sh.at[pl.ds(r0 + rep * _K, _K)])
        plsc.subcore_barrier()

        def pair_body(t, carry):
            ia = 2 * t
            ib = ia + 1
            da = pltpu.async_copy(g_hbm.at[idxs_v.at[ia]], rows_v, sem)
            db = pltpu.async_copy(g_hbm.at[idxs_v.at[ib]], rows_b, sem_b)
            da.wait()
            pltpu.sync_copy(rows_v, acc_sh.at[idxd_v.at[ia]], add=True)
            db.wait()
            pltpu.sync_copy(rows_b, acc_sh.at[idxd_v.at[ib]], add=True)
            return carry

        for half in range(2):
            pltpu.sync_copy(src_hbm.at[w, half], idxs_v)
            pltpu.sync_copy(dst_hbm.at[w, half], idxd_v)
            lax.fori_loop(0, _HC // 2, pair_body, 0)

        plsc.subcore_barrier()
        for rep in range(_RPT // _K):
            rr = r0 + rep * _K
            pltpu.sync_copy(acc_sh.at[pl.ds(rr, _K)], rows_v)
            pltpu.sync_copy(rows_v, out_hbm.at[c, pl.ds(rr, _K)])

    return spmm_kernel(g, srcp, dstp)


def _tc_input(xp, counts_t, w_in, b_in):
    def body(x_ref, cnt_ref, w_ref, b_ref, h0_ref, g_ref, dis_ref):
        h = jnp.dot(x_ref[...], w_ref[...], preferred_element_type=jnp.float32)
        h = jnp.maximum(h + b_ref[...], 0.0)
        deg = 1.0 + cnt_ref[:, 0:1] + cnt_ref[:, 1:2]
        dis = lax.rsqrt(deg)
        h0_ref[...] = h
        g_ref[...] = h * dis
        dis_ref[...] = dis

    return pl.pallas_call(
        body,
        grid=(_GRID,),
        in_specs=[
            pl.BlockSpec((_BR, _D), lambda i: (i, 0)),
            pl.BlockSpec((_BR, 2), lambda i: (i, 0)),
            pl.BlockSpec((_D, _H), lambda i: (0, 0)),
            pl.BlockSpec((1, _H), lambda i: (0, 0)),
        ],
        out_specs=[
            pl.BlockSpec((_BR, _H), lambda i: (i, 0)),
            pl.BlockSpec((_BR, _H), lambda i: (i, 0)),
            pl.BlockSpec((_BR, 1), lambda i: (i, 0)),
        ],
        out_shape=[
            jax.ShapeDtypeStruct((_NP, _H), jnp.float32),
            jax.ShapeDtypeStruct((_NP, _H), jnp.float32),
            jax.ShapeDtypeStruct((_NP, 1), jnp.float32),
        ],
    )(xp, counts_t, w_in, b_in)


def _tc_layer(acc, h0, g, dis, w, b, beta_arr):
    def body(beta_ref, acc_ref, h0_ref, g_ref, dis_ref, w_ref, b_ref,
             h_ref, gout_ref):
        asum = acc_ref[0] + acc_ref[1] + g_ref[...]
        dis_b = dis_ref[...]
        hi = asum * dis_b
        support = (1.0 - _ALPHA) * hi + _ALPHA * h0_ref[...]
        t = jnp.dot(support, w_ref[...], preferred_element_type=jnp.float32)
        beta = beta_ref[0]
        out = beta * t + (1.0 - beta) * support + b_ref[...]
        h = jnp.maximum(out, 0.0)
        h_ref[...] = h
        gout_ref[...] = h * dis_b

    return pl.pallas_call(
        body,
        grid=(_GRID,),
        in_specs=[
            pl.BlockSpec(memory_space=pltpu.SMEM),
            pl.BlockSpec((_NC, _BR, _H), lambda i: (0, i, 0)),
            pl.BlockSpec((_BR, _H), lambda i: (i, 0)),
            pl.BlockSpec((_BR, _H), lambda i: (i, 0)),
            pl.BlockSpec((_BR, 1), lambda i: (i, 0)),
            pl.BlockSpec((_H, _H), lambda i: (0, 0)),
            pl.BlockSpec((1, _H), lambda i: (0, 0)),
        ],
        out_specs=[
            pl.BlockSpec((_BR, _H), lambda i: (i, 0)),
            pl.BlockSpec((_BR, _H), lambda i: (i, 0)),
        ],
        out_shape=[
            jax.ShapeDtypeStruct((_NP, _H), jnp.float32),
            jax.ShapeDtypeStruct((_NP, _H), jnp.float32),
        ],
    )(beta_arr, acc, h0, g, dis, w, b)


def _tc_out(h, w_out, b_out):
    grid = -(-_N // _BR)

    def body(h_ref, w_ref, b_ref, o_ref):
        o_ref[...] = (
            jnp.dot(h_ref[...], w_ref[...], preferred_element_type=jnp.float32)
            + b_ref[...]
        )

    return pl.pallas_call(
        body,
        grid=(grid,),
        in_specs=[
            pl.BlockSpec((_BR, _H), lambda i: (i, 0)),
            pl.BlockSpec((_H, _C), lambda i: (0, 0)),
            pl.BlockSpec((1, _C), lambda i: (0, 0)),
        ],
        out_specs=pl.BlockSpec((_BR, _C), lambda i: (i, 0)),
        out_shape=jax.ShapeDtypeStruct((_N, _C), jnp.float32),
    )(h, w_out, b_out)


def kernel(x, edge_index, W_in, b_in, Wl, bl, W_out, b_out):
    src = edge_index[0]
    dst = edge_index[1]
    pad = _N + (jnp.arange(_EP - _E, dtype=jnp.int32) % (_NP - _N))
    srcp = jnp.concatenate([src, pad])
    dstp = jnp.concatenate([dst, pad])
    xp = jnp.pad(x, ((0, _NP - _N), (0, 0)))

    src4 = srcp.reshape(_NT, 2, _CHUNKS // 2, _K)
    dst4 = dstp.reshape(_NT, 2, _CHUNKS // 2, _K)

    counts = _sc_degree(srcp)
    h0, g, dis = _tc_input(xp, counts.T, W_in, b_in.reshape(1, _H))
    h = h0
    for i in range(_L):
        beta = math.log(_LAMDA / (i + 1) + 1.0)
        acc = _sc_spmm(g, src4, dst4)
        h, g = _tc_layer(acc, h0, g, dis, Wl[i], bl[i].reshape(1, _H),
                         jnp.array([beta], jnp.float32))
    return _tc_out(h, W_out, b_out.reshape(1, _C))


# speedup vs baseline: 3.6761x; 1.0326x over previous
import functools
import math

import jax
import jax.numpy as jnp
from jax import lax
from jax.experimental import pallas as pl
from jax.experimental.pallas import tpu as pltpu
from jax.experimental.pallas import tpu_sc as plsc

_N = 10000
_E = 320000
_D = 128
_H = 128
_C = 40
_L = 8
_ALPHA = 0.1
_LAMDA = 0.5

_NC = 2
_NS = 16
_NT = _NC * _NS

_NP = 10240
_RPT = _NP // _NS
_K = 128
_G = 16
_GROUPS = 5
_CHUNKS = _G * _GROUPS
_EP = _NT * _CHUNKS * _K

_BR = 512
_GRID = _NP // _BR


def _mesh():
    return plsc.VectorSubcoreMesh(core_axis_name="c", subcore_axis_name="s")


def _sc_degree(srcp):
    @functools.partial(
        pl.kernel,
        out_type=jax.ShapeDtypeStruct((_NC, _NP), jnp.float32),
        mesh=_mesh(),
        scratch_types=[
            pltpu.VMEM_SHARED((_NP,), jnp.float32),
            pltpu.VMEM((_K,), jnp.int32),
            pltpu.VMEM((_K,), jnp.float32),
            pltpu.VMEM((_RPT,), jnp.float32),
        ],
    )
    def deg_kernel(src_hbm, out_hbm, cnt_sh, idx_v, ones_v, bounce_v):
        c = lax.axis_index("c")
        s = lax.axis_index("s")
        w = s * _NC + c
        for j in range(_K // 16):
            ones_v[pl.ds(j * 16, 16)] = jnp.full((16,), 1.0, jnp.float32)
        for j in range(_RPT // 16):
            bounce_v[pl.ds(j * 16, 16)] = jnp.zeros((16,), jnp.float32)
        pltpu.sync_copy(bounce_v, cnt_sh.at[pl.ds(s * _RPT, _RPT)])
        plsc.subcore_barrier()
        base = w * (_CHUNKS * _K)

        def body(j, carry):
            off = base + j * _K
            pltpu.sync_copy(src_hbm.at[pl.ds(off, _K)], idx_v)
            pltpu.sync_copy(ones_v, cnt_sh.at[idx_v], add=True)
            return carry

        lax.fori_loop(0, _CHUNKS, body, 0)
        plsc.subcore_barrier()
        pltpu.sync_copy(cnt_sh.at[pl.ds(s * _RPT, _RPT)], bounce_v)
        pltpu.sync_copy(bounce_v, out_hbm.at[c, pl.ds(s * _RPT, _RPT)])

    return deg_kernel(srcp)


def _sc_spmm(g, srcp, dstp):
    _HC = _CHUNKS // 2

    @functools.partial(
        pl.kernel,
        out_type=jax.ShapeDtypeStruct((_NC, _NP, _H), jnp.float32),
        mesh=_mesh(),
        scratch_types=[
            pltpu.VMEM_SHARED((_NP, _H), jnp.float32),
            pltpu.VMEM((_HC, _K), jnp.int32),
            pltpu.VMEM((_HC, _K), jnp.int32),
            pltpu.VMEM((_K, _H), jnp.float32),
            pltpu.VMEM((_K, _H), jnp.float32),
            pltpu.SemaphoreType.DMA,
            pltpu.SemaphoreType.DMA,
            pltpu.SemaphoreType.DMA,
            pltpu.SemaphoreType.DMA,
        ],
    )
    def spmm_kernel(g_hbm, src_hbm, dst_hbm, out_hbm, acc_sh,
                    idxs_v, idxd_v, rows_v, rows_b, sem, sem_b,
                    ssem_a, ssem_b):
        c = lax.axis_index("c")
        s = lax.axis_index("s")
        w = s * _NC + c

        def zbody(i, carry):
            for j in range(_H // 16):
                rows_v[i, pl.ds(j * 16, 16)] = jnp.zeros((16,), jnp.float32)
            return carry

        lax.fori_loop(0, _K, zbody, 0)
        r0 = s * _RPT
        for rep in range(_RPT // _K):
            pltpu.sync_copy(rows_v, acc_sh.at[pl.ds(r0 + rep * _K, _K)])
        plsc.subcore_barrier()

        def pair_body(t, carry):
            ia = 2 * t
            ib = ia + 1

            @pl.when(t > 0)
            def _():
                pltpu.make_async_copy(rows_v, acc_sh.at[idxd_v.at[ia]],
                                      ssem_a).wait()

            da = pltpu.async_copy(g_hbm.at[idxs_v.at[ia]], rows_v, sem)

            @pl.when(t > 0)
            def _():
                pltpu.make_async_copy(rows_b, acc_sh.at[idxd_v.at[ib]],
                                      ssem_b).wait()

            db = pltpu.async_copy(g_hbm.at[idxs_v.at[ib]], rows_b, sem_b)
            da.wait()
            pltpu.make_async_copy(rows_v, acc_sh.at[idxd_v.at[ia]],
                                  ssem_a).start(add=True)
            db.wait()
            pltpu.make_async_copy(rows_b, acc_sh.at[idxd_v.at[ib]],
                                  ssem_b).start(add=True)
            return carry

        for half in range(2):
            pltpu.sync_copy(src_hbm.at[w, half], idxs_v)
            pltpu.sync_copy(dst_hbm.at[w, half], idxd_v)
            lax.fori_loop(0, _HC // 2, pair_body, 0)
            pltpu.make_async_copy(rows_v, acc_sh.at[idxd_v.at[0]],
                                  ssem_a).wait()
            pltpu.make_async_copy(rows_b, acc_sh.at[idxd_v.at[0]],
                                  ssem_b).wait()

        plsc.subcore_barrier()
        for rep in range(_RPT // _K):
            rr = r0 + rep * _K
            pltpu.sync_copy(acc_sh.at[pl.ds(rr, _K)], rows_v)
            pltpu.sync_copy(rows_v, out_hbm.at[c, pl.ds(rr, _K)])

    return spmm_kernel(g, srcp, dstp)


def _tc_input(xp, counts_t, w_in, b_in):
    def body(x_ref, cnt_ref, w_ref, b_ref, h0_ref, g_ref, dis_ref):
        h = jnp.dot(x_ref[...], w_ref[...], preferred_element_type=jnp.float32)
        h = jnp.maximum(h + b_ref[...], 0.0)
        deg = 1.0 + cnt_ref[:, 0:1] + cnt_ref[:, 1:2]
        dis = lax.rsqrt(deg)
        h0_ref[...] = h
        g_ref[...] = h * dis
        dis_ref[...] = dis

    return pl.pallas_call(
        body,
        grid=(_GRID,),
        in_specs=[
            pl.BlockSpec((_BR, _D), lambda i: (i, 0)),
            pl.BlockSpec((_BR, 2), lambda i: (i, 0)),
            pl.BlockSpec((_D, _H), lambda i: (0, 0)),
            pl.BlockSpec((1, _H), lambda i: (0, 0)),
        ],
        out_specs=[
            pl.BlockSpec((_BR, _H), lambda i: (i, 0)),
            pl.BlockSpec((_BR, _H), lambda i: (i, 0)),
            pl.BlockSpec((_BR, 1), lambda i: (i, 0)),
        ],
        out_shape=[
            jax.ShapeDtypeStruct((_NP, _H), jnp.float32),
            jax.ShapeDtypeStruct((_NP, _H), jnp.float32),
            jax.ShapeDtypeStruct((_NP, 1), jnp.float32),
        ],
    )(xp, counts_t, w_in, b_in)


def _tc_layer(acc, h0, g, dis, w, b, beta_arr):
    def body(beta_ref, acc_ref, h0_ref, g_ref, dis_ref, w_ref, b_ref,
             h_ref, gout_ref):
        asum = acc_ref[0] + acc_ref[1] + g_ref[...]
        dis_b = dis_ref[...]
        hi = asum * dis_b
        support = (1.0 - _ALPHA) * hi + _ALPHA * h0_ref[...]
        t = jnp.dot(support, w_ref[...], preferred_element_type=jnp.float32)
        beta = beta_ref[0]
        out = beta * t + (1.0 - beta) * support + b_ref[...]
        h = jnp.maximum(out, 0.0)
        h_ref[...] = h
        gout_ref[...] = h * dis_b

    return pl.pallas_call(
        body,
        grid=(_GRID,),
        in_specs=[
            pl.BlockSpec(memory_space=pltpu.SMEM),
            pl.BlockSpec((_NC, _BR, _H), lambda i: (0, i, 0)),
            pl.BlockSpec((_BR, _H), lambda i: (i, 0)),
            pl.BlockSpec((_BR, _H), lambda i: (i, 0)),
            pl.BlockSpec((_BR, 1), lambda i: (i, 0)),
            pl.BlockSpec((_H, _H), lambda i: (0, 0)),
            pl.BlockSpec((1, _H), lambda i: (0, 0)),
        ],
        out_specs=[
            pl.BlockSpec((_BR, _H), lambda i: (i, 0)),
            pl.BlockSpec((_BR, _H), lambda i: (i, 0)),
        ],
        out_shape=[
            jax.ShapeDtypeStruct((_NP, _H), jnp.float32),
            jax.ShapeDtypeStruct((_NP, _H), jnp.float32),
        ],
    )(beta_arr, acc, h0, g, dis, w, b)


def _tc_out(h, w_out, b_out):
    grid = -(-_N // _BR)

    def body(h_ref, w_ref, b_ref, o_ref):
        o_ref[...] = (
            jnp.dot(h_ref[...], w_ref[...], preferred_element_type=jnp.float32)
            + b_ref[...]
        )

    return pl.pallas_call(
        body,
        grid=(grid,),
        in_specs=[
            pl.BlockSpec((_BR, _H), lambda i: (i, 0)),
            pl.BlockSpec((_H, _C), lambda i: (0, 0)),
            pl.BlockSpec((1, _C), lambda i: (0, 0)),
        ],
        out_specs=pl.BlockSpec((_BR, _C), lambda i: (i, 0)),
        out_shape=jax.ShapeDtypeStruct((_N, _C), jnp.float32),
    )(h, w_out, b_out)


def kernel(x, edge_index, W_in, b_in, Wl, bl, W_out, b_out):
    src = edge_index[0]
    dst = edge_index[1]
    pad = _N + (jnp.arange(_EP - _E, dtype=jnp.int32) % (_NP - _N))
    srcp = jnp.concatenate([src, pad])
    dstp = jnp.concatenate([dst, pad])
    xp = jnp.pad(x, ((0, _NP - _N), (0, 0)))

    src4 = srcp.reshape(_NT, 2, _CHUNKS // 2, _K)
    dst4 = dstp.reshape(_NT, 2, _CHUNKS // 2, _K)

    counts = _sc_degree(srcp)
    h0, g, dis = _tc_input(xp, counts.T, W_in, b_in.reshape(1, _H))
    h = h0
    for i in range(_L):
        beta = math.log(_LAMDA / (i + 1) + 1.0)
        acc = _sc_spmm(g, src4, dst4)
        h, g = _tc_layer(acc, h0, g, dis, Wl[i], bl[i].reshape(1, _H),
                         jnp.array([beta], jnp.float32))
    return _tc_out(h, W_out, b_out.reshape(1, _C))


# speedup vs baseline: 3.7643x; 1.0240x over previous
import functools
import math

import jax
import jax.numpy as jnp
from jax import lax
from jax.experimental import pallas as pl
from jax.experimental.pallas import tpu as pltpu
from jax.experimental.pallas import tpu_sc as plsc

_N = 10000
_E = 320000
_D = 128
_H = 128
_C = 40
_L = 8
_ALPHA = 0.1
_LAMDA = 0.5

_NC = 2
_NS = 16
_NT = _NC * _NS

_NP = 10240
_RPT = _NP // _NS
_K = 128
_G = 16
_GROUPS = 5
_CHUNKS = _G * _GROUPS
_EP = _NT * _CHUNKS * _K

_BR = 512
_GRID = _NP // _BR


def _mesh():
    return plsc.VectorSubcoreMesh(core_axis_name="c", subcore_axis_name="s")


def _sc_degree(srcp):
    @functools.partial(
        pl.kernel,
        out_type=jax.ShapeDtypeStruct((_NC, _NP), jnp.float32),
        mesh=_mesh(),
        scratch_types=[
            pltpu.VMEM_SHARED((_NP,), jnp.float32),
            pltpu.VMEM((_K,), jnp.int32),
            pltpu.VMEM((_K,), jnp.float32),
            pltpu.VMEM((_RPT,), jnp.float32),
        ],
    )
    def deg_kernel(src_hbm, out_hbm, cnt_sh, idx_v, ones_v, bounce_v):
        c = lax.axis_index("c")
        s = lax.axis_index("s")
        w = s * _NC + c
        for j in range(_K // 16):
            ones_v[pl.ds(j * 16, 16)] = jnp.full((16,), 1.0, jnp.float32)
        for j in range(_RPT // 16):
            bounce_v[pl.ds(j * 16, 16)] = jnp.zeros((16,), jnp.float32)
        pltpu.sync_copy(bounce_v, cnt_sh.at[pl.ds(s * _RPT, _RPT)])
        plsc.subcore_barrier()
        base = w * (_CHUNKS * _K)

        def body(j, carry):
            off = base + j * _K
            pltpu.sync_copy(src_hbm.at[pl.ds(off, _K)], idx_v)
            pltpu.sync_copy(ones_v, cnt_sh.at[idx_v], add=True)
            return carry

        lax.fori_loop(0, _CHUNKS, body, 0)
        plsc.subcore_barrier()
        pltpu.sync_copy(cnt_sh.at[pl.ds(s * _RPT, _RPT)], bounce_v)
        pltpu.sync_copy(bounce_v, out_hbm.at[c, pl.ds(s * _RPT, _RPT)])

    return deg_kernel(srcp)


def _sc_spmm(g, srcp, dstp):
    _HC = _CHUNKS // 2

    @functools.partial(
        pl.kernel,
        out_type=jax.ShapeDtypeStruct((_NC, _NP, _H), jnp.float32),
        mesh=_mesh(),
        scratch_types=[
            pltpu.VMEM_SHARED((_NP, _H), jnp.float32),
            pltpu.VMEM((_HC, _K), jnp.int32),
            pltpu.VMEM((_HC, _K), jnp.int32),
            pltpu.VMEM((_K, _H), jnp.float32),
            pltpu.VMEM((_K, _H), jnp.float32),
            pltpu.SemaphoreType.DMA,
            pltpu.SemaphoreType.DMA,
            pltpu.SemaphoreType.DMA,
            pltpu.SemaphoreType.DMA,
        ],
    )
    def spmm_kernel(g_hbm, src_hbm, dst_hbm, out_hbm, acc_sh,
                    idxs_v, idxd_v, rows_v, rows_b, sem, sem_b,
                    ssem_a, ssem_b):
        c = lax.axis_index("c")
        s = lax.axis_index("s")
        w = s * _NC + c

        di1 = pltpu.async_copy(src_hbm.at[w, 0], idxs_v, sem)
        di2 = pltpu.async_copy(dst_hbm.at[w, 0], idxd_v, sem_b)

        def zbody(i, carry):
            for j in range(_H // 16):
                rows_v[i, pl.ds(j * 16, 16)] = jnp.zeros((16,), jnp.float32)
            return carry

        lax.fori_loop(0, _K, zbody, 0)
        r0 = s * _RPT
        zd = []
        for rep in range(_RPT // _K):
            d = pltpu.make_async_copy(
                rows_v, acc_sh.at[pl.ds(r0 + rep * _K, _K)], ssem_a)
            d.start()
            zd.append(d)
        for d in zd:
            d.wait()
        di1.wait()
        di2.wait()
        plsc.subcore_barrier()

        def pair_body(t, carry):
            ia = 2 * t
            ib = ia + 1

            @pl.when(t > 0)
            def _():
                pltpu.make_async_copy(rows_v, acc_sh.at[idxd_v.at[ia]],
                                      ssem_a).wait()

            da = pltpu.async_copy(g_hbm.at[idxs_v.at[ia]], rows_v, sem)

            @pl.when(t > 0)
            def _():
                pltpu.make_async_copy(rows_b, acc_sh.at[idxd_v.at[ib]],
                                      ssem_b).wait()

            db = pltpu.async_copy(g_hbm.at[idxs_v.at[ib]], rows_b, sem_b)
            da.wait()
            pltpu.make_async_copy(rows_v, acc_sh.at[idxd_v.at[ia]],
                                  ssem_a).start(add=True)
            db.wait()
            pltpu.make_async_copy(rows_b, acc_sh.at[idxd_v.at[ib]],
                                  ssem_b).start(add=True)
            return carry

        for half in range(2):
            if half == 1:
                pltpu.sync_copy(src_hbm.at[w, half], idxs_v)
                pltpu.sync_copy(dst_hbm.at[w, half], idxd_v)
            lax.fori_loop(0, _HC // 2, pair_body, 0)
            pltpu.make_async_copy(rows_v, acc_sh.at[idxd_v.at[0]],
                                  ssem_a).wait()
            pltpu.make_async_copy(rows_b, acc_sh.at[idxd_v.at[0]],
                                  ssem_b).wait()

        plsc.subcore_barrier()
        wdescs = [None, None]
        for rep in range(_RPT // _K):
            rr = r0 + rep * _K
            buf = rows_v if rep % 2 == 0 else rows_b
            wsem = sem if rep % 2 == 0 else sem_b
            if wdescs[rep % 2] is not None:
                wdescs[rep % 2].wait()
            pltpu.sync_copy(acc_sh.at[pl.ds(rr, _K)], buf)
            d = pltpu.make_async_copy(buf, out_hbm.at[c, pl.ds(rr, _K)], wsem)
            d.start()
            wdescs[rep % 2] = d
        for d in wdescs:
            d.wait()

    return spmm_kernel(g, srcp, dstp)


def _tc_input(xp, counts_t, w_in, b_in):
    def body(x_ref, cnt_ref, w_ref, b_ref, h0_ref, g_ref, dis_ref):
        h = jnp.dot(x_ref[...], w_ref[...], preferred_element_type=jnp.float32)
        h = jnp.maximum(h + b_ref[...], 0.0)
        deg = 1.0 + cnt_ref[:, 0:1] + cnt_ref[:, 1:2]
        dis = lax.rsqrt(deg)
        h0_ref[...] = h
        g_ref[...] = h * dis
        dis_ref[...] = dis

    return pl.pallas_call(
        body,
        grid=(_GRID,),
        in_specs=[
            pl.BlockSpec((_BR, _D), lambda i: (i, 0)),
            pl.BlockSpec((_BR, 2), lambda i: (i, 0)),
            pl.BlockSpec((_D, _H), lambda i: (0, 0)),
            pl.BlockSpec((1, _H), lambda i: (0, 0)),
        ],
        out_specs=[
            pl.BlockSpec((_BR, _H), lambda i: (i, 0)),
            pl.BlockSpec((_BR, _H), lambda i: (i, 0)),
            pl.BlockSpec((_BR, 1), lambda i: (i, 0)),
        ],
        out_shape=[
            jax.ShapeDtypeStruct((_NP, _H), jnp.float32),
            jax.ShapeDtypeStruct((_NP, _H), jnp.float32),
            jax.ShapeDtypeStruct((_NP, 1), jnp.float32),
        ],
    )(xp, counts_t, w_in, b_in)


def _tc_layer(acc, h0, g, dis, w, b, beta_arr):
    def body(beta_ref, acc_ref, h0_ref, g_ref, dis_ref, w_ref, b_ref,
             h_ref, gout_ref):
        asum = acc_ref[0] + acc_ref[1] + g_ref[...]
        dis_b = dis_ref[...]
        hi = asum * dis_b
        support = (1.0 - _ALPHA) * hi + _ALPHA * h0_ref[...]
        t = jnp.dot(support, w_ref[...], preferred_element_type=jnp.float32)
        beta = beta_ref[0]
        out = beta * t + (1.0 - beta) * support + b_ref[...]
        h = jnp.maximum(out, 0.0)
        h_ref[...] = h
        gout_ref[...] = h * dis_b

    return pl.pallas_call(
        body,
        grid=(_GRID,),
        in_specs=[
            pl.BlockSpec(memory_space=pltpu.SMEM),
            pl.BlockSpec((_NC, _BR, _H), lambda i: (0, i, 0)),
            pl.BlockSpec((_BR, _H), lambda i: (i, 0)),
            pl.BlockSpec((_BR, _H), lambda i: (i, 0)),
            pl.BlockSpec((_BR, 1), lambda i: (i, 0)),
            pl.BlockSpec((_H, _H), lambda i: (0, 0)),
            pl.BlockSpec((1, _H), lambda i: (0, 0)),
        ],
        out_specs=[
            pl.BlockSpec((_BR, _H), lambda i: (i, 0)),
            pl.BlockSpec((_BR, _H), lambda i: (i, 0)),
        ],
        out_shape=[
            jax.ShapeDtypeStruct((_NP, _H), jnp.float32),
            jax.ShapeDtypeStruct((_NP, _H), jnp.float32),
        ],
    )(beta_arr, acc, h0, g, dis, w, b)


def _tc_out(h, w_out, b_out):
    grid = -(-_N // _BR)

    def body(h_ref, w_ref, b_ref, o_ref):
        o_ref[...] = (
            jnp.dot(h_ref[...], w_ref[...], preferred_element_type=jnp.float32)
            + b_ref[...]
        )

    return pl.pallas_call(
        body,
        grid=(grid,),
        in_specs=[
            pl.BlockSpec((_BR, _H), lambda i: (i, 0)),
            pl.BlockSpec((_H, _C), lambda i: (0, 0)),
            pl.BlockSpec((1, _C), lambda i: (0, 0)),
        ],
        out_specs=pl.BlockSpec((_BR, _C), lambda i: (i, 0)),
        out_shape=jax.ShapeDtypeStruct((_N, _C), jnp.float32),
    )(h, w_out, b_out)


def kernel(x, edge_index, W_in, b_in, Wl, bl, W_out, b_out):
    src = edge_index[0]
    dst = edge_index[1]
    pad = _N + (jnp.arange(_EP - _E, dtype=jnp.int32) % (_NP - _N))
    srcp = jnp.concatenate([src, pad])
    dstp = jnp.concatenate([dst, pad])
    xp = jnp.pad(x, ((0, _NP - _N), (0, 0)))

    src4 = srcp.reshape(_NT, 2, _CHUNKS // 2, _K)
    dst4 = dstp.reshape(_NT, 2, _CHUNKS // 2, _K)

    counts = _sc_degree(srcp)
    h0, g, dis = _tc_input(xp, counts.T, W_in, b_in.reshape(1, _H))
    h = h0
    for i in range(_L):
        beta = math.log(_LAMDA / (i + 1) + 1.0)
        acc = _sc_spmm(g, src4, dst4)
        h, g = _tc_layer(acc, h0, g, dis, Wl[i], bl[i].reshape(1, _H),
                         jnp.array([beta], jnp.float32))
    return _tc_out(h, W_out, b_out.reshape(1, _C))


# speedup vs baseline: 3.7839x; 1.0052x over previous
import functools
import math

import jax
import jax.numpy as jnp
from jax import lax
from jax.experimental import pallas as pl
from jax.experimental.pallas import tpu as pltpu
from jax.experimental.pallas import tpu_sc as plsc

_N = 10000
_E = 320000
_D = 128
_H = 128
_C = 40
_L = 8
_ALPHA = 0.1
_LAMDA = 0.5

_NC = 2
_NS = 16
_NT = _NC * _NS

_NP = 10240
_RPT = _NP // _NS
_K = 128
_G = 16
_GROUPS = 5
_CHUNKS = _G * _GROUPS
_EP = _NT * _CHUNKS * _K

_BR = 512
_GRID = _NP // _BR


def _mesh():
    return plsc.VectorSubcoreMesh(core_axis_name="c", subcore_axis_name="s")


def _sc_degree(srcp):
    @functools.partial(
        pl.kernel,
        out_type=jax.ShapeDtypeStruct((_NC, _NP), jnp.float32),
        mesh=_mesh(),
        scratch_types=[
            pltpu.VMEM_SHARED((_NP,), jnp.float32),
            pltpu.VMEM((_K,), jnp.int32),
            pltpu.VMEM((_K,), jnp.float32),
            pltpu.VMEM((_RPT,), jnp.float32),
        ],
    )
    def deg_kernel(src_hbm, out_hbm, cnt_sh, idx_v, ones_v, bounce_v):
        c = lax.axis_index("c")
        s = lax.axis_index("s")
        w = s * _NC + c
        for j in range(_K // 16):
            ones_v[pl.ds(j * 16, 16)] = jnp.full((16,), 1.0, jnp.float32)
        for j in range(_RPT // 16):
            bounce_v[pl.ds(j * 16, 16)] = jnp.zeros((16,), jnp.float32)
        pltpu.sync_copy(bounce_v, cnt_sh.at[pl.ds(s * _RPT, _RPT)])
        plsc.subcore_barrier()
        base = w * (_CHUNKS * _K)

        def body(j, carry):
            off = base + j * _K
            pltpu.sync_copy(src_hbm.at[pl.ds(off, _K)], idx_v)
            pltpu.sync_copy(ones_v, cnt_sh.at[idx_v], add=True)
            return carry

        lax.fori_loop(0, _CHUNKS, body, 0)
        plsc.subcore_barrier()
        pltpu.sync_copy(cnt_sh.at[pl.ds(s * _RPT, _RPT)], bounce_v)
        pltpu.sync_copy(bounce_v, out_hbm.at[c, pl.ds(s * _RPT, _RPT)])

    return deg_kernel(srcp)


def _sc_spmm(g, srcp, dstp):
    _HC = _CHUNKS // 2

    @functools.partial(
        pl.kernel,
        out_type=jax.ShapeDtypeStruct((_NC, _NP, _H), jnp.float32),
        mesh=_mesh(),
        scratch_types=[
            pltpu.VMEM_SHARED((_NP, _H), jnp.float32),
            pltpu.VMEM((_HC, _K), jnp.int32),
            pltpu.VMEM((_HC, _K), jnp.int32),
            pltpu.VMEM((_K, _H), jnp.float32),
            pltpu.VMEM((_K, _H), jnp.float32),
            pltpu.SemaphoreType.DMA,
            pltpu.SemaphoreType.DMA,
            pltpu.SemaphoreType.DMA,
            pltpu.SemaphoreType.DMA,
        ],
    )
    def spmm_kernel(g_hbm, src_hbm, dst_hbm, out_hbm, acc_sh,
                    idxs_v, idxd_v, rows_v, rows_b, sem, sem_b,
                    ssem_a, ssem_b):
        c = lax.axis_index("c")
        s = lax.axis_index("s")
        w = s * _NC + c

        di1 = pltpu.async_copy(src_hbm.at[w, 0], idxs_v, sem)
        di2 = pltpu.async_copy(dst_hbm.at[w, 0], idxd_v, sem_b)

        def zbody(i, carry):
            for j in range(_H // 16):
                rows_v[i, pl.ds(j * 16, 16)] = jnp.zeros((16,), jnp.float32)
            return carry

        lax.fori_loop(0, _K, zbody, 0)
        r0 = s * _RPT
        zd = []
        for rep in range(_RPT // _K):
            d = pltpu.make_async_copy(
                rows_v, acc_sh.at[pl.ds(r0 + rep * _K, _K)], ssem_a)
            d.start()
            zd.append(d)
        for d in zd:
            d.wait()
        di1.wait()
        di2.wait()
        plsc.subcore_barrier()

        def pair_body(t, carry):
            ia = 2 * t
            ib = ia + 1

            @pl.when(t > 0)
            def _():
                pltpu.make_async_copy(rows_v, acc_sh.at[idxd_v.at[ia]],
                                      ssem_a).wait()

            da = pltpu.async_copy(g_hbm.at[idxs_v.at[ia]], rows_v, sem)

            @pl.when(t > 0)
            def _():
                pltpu.make_async_copy(rows_b, acc_sh.at[idxd_v.at[ib]],
                                      ssem_b).wait()

            db = pltpu.async_copy(g_hbm.at[idxs_v.at[ib]], rows_b, sem_b)
            da.wait()
            pltpu.make_async_copy(rows_v, acc_sh.at[idxd_v.at[ia]],
                                  ssem_a).start(add=True)
            db.wait()
            pltpu.make_async_copy(rows_b, acc_sh.at[idxd_v.at[ib]],
                                  ssem_b).start(add=True)
            return carry

        for half in range(2):
            if half == 1:
                pltpu.sync_copy(src_hbm.at[w, half], idxs_v)
                pltpu.sync_copy(dst_hbm.at[w, half], idxd_v)
            lax.fori_loop(0, _HC // 2, pair_body, 0)
            pltpu.make_async_copy(rows_v, acc_sh.at[idxd_v.at[0]],
                                  ssem_a).wait()
            pltpu.make_async_copy(rows_b, acc_sh.at[idxd_v.at[0]],
                                  ssem_b).wait()

        plsc.subcore_barrier()
        wdescs = [None, None]
        for rep in range(_RPT // _K):
            rr = r0 + rep * _K
            buf = rows_v if rep % 2 == 0 else rows_b
            wsem = sem if rep % 2 == 0 else sem_b
            if wdescs[rep % 2] is not None:
                wdescs[rep % 2].wait()
            pltpu.sync_copy(acc_sh.at[pl.ds(rr, _K)], buf)
            d = pltpu.make_async_copy(buf, out_hbm.at[c, pl.ds(rr, _K)], wsem)
            d.start()
            wdescs[rep % 2] = d
        for d in wdescs:
            d.wait()

    return spmm_kernel(g, srcp, dstp)


def _tc_input(xp, counts_t, w_in, b_in):
    def body(x_ref, cnt_ref, w_ref, b_ref, h0_ref, g_ref, dis_ref):
        h = jnp.dot(x_ref[...], w_ref[...], preferred_element_type=jnp.float32)
        h = jnp.maximum(h + b_ref[...], 0.0)
        deg = 1.0 + cnt_ref[:, 0:1] + cnt_ref[:, 1:2]
        dis = lax.rsqrt(deg)
        h0_ref[...] = h
        g_ref[...] = h * dis
        dis_ref[...] = dis

    return pl.pallas_call(
        body,
        grid=(_GRID,),
        in_specs=[
            pl.BlockSpec((_BR, _D), lambda i: (i, 0)),
            pl.BlockSpec((_BR, 2), lambda i: (i, 0)),
            pl.BlockSpec((_D, _H), lambda i: (0, 0)),
            pl.BlockSpec((1, _H), lambda i: (0, 0)),
        ],
        out_specs=[
            pl.BlockSpec((_BR, _H), lambda i: (i, 0)),
            pl.BlockSpec((_BR, _H), lambda i: (i, 0)),
            pl.BlockSpec((_BR, 1), lambda i: (i, 0)),
        ],
        out_shape=[
            jax.ShapeDtypeStruct((_NP, _H), jnp.float32),
            jax.ShapeDtypeStruct((_NP, _H), jnp.float32),
            jax.ShapeDtypeStruct((_NP, 1), jnp.float32),
        ],
    )(xp, counts_t, w_in, b_in)


def _tc_layer(acc, h0, g, dis, w, b, beta_arr):
    def body(beta_ref, acc_ref, h0_ref, g_ref, dis_ref, w_ref, b_ref,
             h_ref, gout_ref):
        asum = acc_ref[0] + acc_ref[1] + g_ref[...]
        dis_b = dis_ref[...]
        hi = asum * dis_b
        support = (1.0 - _ALPHA) * hi + _ALPHA * h0_ref[...]
        t = jnp.dot(support, w_ref[...], preferred_element_type=jnp.float32)
        beta = beta_ref[0]
        out = beta * t + (1.0 - beta) * support + b_ref[...]
        h = jnp.maximum(out, 0.0)
        h_ref[...] = h
        gout_ref[...] = h * dis_b

    return pl.pallas_call(
        body,
        grid=(_GRID,),
        in_specs=[
            pl.BlockSpec(memory_space=pltpu.SMEM),
            pl.BlockSpec((_NC, _BR, _H), lambda i: (0, i, 0)),
            pl.BlockSpec((_BR, _H), lambda i: (i, 0)),
            pl.BlockSpec((_BR, _H), lambda i: (i, 0)),
            pl.BlockSpec((_BR, 1), lambda i: (i, 0)),
            pl.BlockSpec((_H, _H), lambda i: (0, 0)),
            pl.BlockSpec((1, _H), lambda i: (0, 0)),
        ],
        out_specs=[
            pl.BlockSpec((_BR, _H), lambda i: (i, 0)),
            pl.BlockSpec((_BR, _H), lambda i: (i, 0)),
        ],
        out_shape=[
            jax.ShapeDtypeStruct((_NP, _H), jnp.float32),
            jax.ShapeDtypeStruct((_NP, _H), jnp.float32),
        ],
    )(beta_arr, acc, h0, g, dis, w, b)


def _tc_layer_out(acc, h0, g, dis, w, b, beta_arr, w_out, b_out):
    def body(beta_ref, acc_ref, h0_ref, g_ref, dis_ref, w_ref, b_ref,
             wo_ref, bo_ref, o_ref):
        asum = acc_ref[0] + acc_ref[1] + g_ref[...]
        hi = asum * dis_ref[...]
        support = (1.0 - _ALPHA) * hi + _ALPHA * h0_ref[...]
        t = jnp.dot(support, w_ref[...], preferred_element_type=jnp.float32)
        beta = beta_ref[0]
        h = jnp.maximum(beta * t + (1.0 - beta) * support + b_ref[...], 0.0)
        o_ref[...] = (
            jnp.dot(h, wo_ref[...], preferred_element_type=jnp.float32)
            + bo_ref[...]
        )

    return pl.pallas_call(
        body,
        grid=(_GRID,),
        in_specs=[
            pl.BlockSpec(memory_space=pltpu.SMEM),
            pl.BlockSpec((_NC, _BR, _H), lambda i: (0, i, 0)),
            pl.BlockSpec((_BR, _H), lambda i: (i, 0)),
            pl.BlockSpec((_BR, _H), lambda i: (i, 0)),
            pl.BlockSpec((_BR, 1), lambda i: (i, 0)),
            pl.BlockSpec((_H, _H), lambda i: (0, 0)),
            pl.BlockSpec((1, _H), lambda i: (0, 0)),
            pl.BlockSpec((_H, _C), lambda i: (0, 0)),
            pl.BlockSpec((1, _C), lambda i: (0, 0)),
        ],
        out_specs=pl.BlockSpec((_BR, _C), lambda i: (i, 0)),
        out_shape=jax.ShapeDtypeStruct((_N, _C), jnp.float32),
    )(beta_arr, acc, h0, g, dis, w, b, w_out, b_out)


def _tc_out(h, w_out, b_out):
    grid = -(-_N // _BR)

    def body(h_ref, w_ref, b_ref, o_ref):
        o_ref[...] = (
            jnp.dot(h_ref[...], w_ref[...], preferred_element_type=jnp.float32)
            + b_ref[...]
        )

    return pl.pallas_call(
        body,
        grid=(grid,),
        in_specs=[
            pl.BlockSpec((_BR, _H), lambda i: (i, 0)),
            pl.BlockSpec((_H, _C), lambda i: (0, 0)),
            pl.BlockSpec((1, _C), lambda i: (0, 0)),
        ],
        out_specs=pl.BlockSpec((_BR, _C), lambda i: (i, 0)),
        out_shape=jax.ShapeDtypeStruct((_N, _C), jnp.float32),
    )(h, w_out, b_out)


def kernel(x, edge_index, W_in, b_in, Wl, bl, W_out, b_out):
    src = edge_index[0]
    dst = edge_index[1]
    pad = _N + (jnp.arange(_EP - _E, dtype=jnp.int32) % (_NP - _N))
    srcp = jnp.concatenate([src, pad])
    dstp = jnp.concatenate([dst, pad])
    xp = jnp.pad(x, ((0, _NP - _N), (0, 0)))

    src4 = srcp.reshape(_NT, 2, _CHUNKS // 2, _K)
    dst4 = dstp.reshape(_NT, 2, _CHUNKS // 2, _K)

    counts = _sc_degree(srcp)
    h0, g, dis = _tc_input(xp, counts.T, W_in, b_in.reshape(1, _H))
    h = h0
    for i in range(_L - 1):
        beta = math.log(_LAMDA / (i + 1) + 1.0)
        acc = _sc_spmm(g, src4, dst4)
        h, g = _tc_layer(acc, h0, g, dis, Wl[i], bl[i].reshape(1, _H),
                         jnp.array([beta], jnp.float32))
    beta = math.log(_LAMDA / _L + 1.0)
    acc = _sc_spmm(g, src4, dst4)
    return _tc_layer_out(acc, h0, g, dis, Wl[_L - 1], bl[_L - 1].reshape(1, _H),
                         jnp.array([beta], jnp.float32),
                         W_out, b_out.reshape(1, _C))
